# baseline (speedup 1.0000x reference)
import functools
import jax
import jax.numpy as jnp
from jax import lax
from jax.experimental import pallas as pl
from jax.experimental.pallas import tpu as pltpu
from jax.experimental.pallas import tpu_sc as plsc

B, N, D = 16, 4096, 256
L = 16
NC, NS = 2, 16
HD = D // 2
G = 128
NBLK = N // G
BPC = B // NC
HV = HD // L
CU = 4
RU = 8
TRASH = N + G
IW = TRASH + L


def _sc_body(x_hbm, mask_hbm, out_hbm, mvec, idxv, ring0, ring1, accv,
             sem0, sem1):
    c = lax.axis_index("c")
    s = lax.axis_index("s")
    batch = c * BPC + s // 2
    h = s % 2
    row0 = batch * N
    hoff = 2 * row0 + h

    pltpu.sync_copy(mask_hbm.at[pl.ds(row0, N)], mvec)

    splat15 = jnp.full((L,), L - 1, jnp.int32)
    iota2 = lax.iota(jnp.int32, L) * 2

    def _compact(i, tv):
        for j in range(CU):
            q = i * CU + j
            mi = mvec[pl.ds(q * L, L)]
            tv = tv + mi
        return tv
    tv = jnp.zeros((L,), jnp.int32) + mvec[pl.ds(0, L)]
    nrows = jnp.sum(tv) >> 4

    fillv = jnp.zeros((L,), jnp.int32) + hoff
    for j in range(G // L):
        idxv[pl.ds(nrows + j * L, L)] = fillv

    for t in range(HV):
        accv[pl.ds(t * L, L)] = jnp.zeros((L,), jnp.float32)

    rings = (ring0, ring1)
    sems = (sem0, sem1)
    nfull = nrows // G
    rem = nrows - nfull * G

    def _start(k):
        pltpu.async_copy(x_hbm.at[idxv.at[pl.ds(k * G, G)]], rings[k % 2],
                         sems[k % 2])

    def _wait(k):
        pltpu.make_async_copy(x_hbm.at[idxv.at[pl.ds(k * G, G)]],
                              rings[k % 2], sems[k % 2]).wait()

    def _accum_full(k):
        buf = rings[k % 2]

        def _rows(r, acc):
            base = r * RU
            for u in range(RU):
                acc = tuple(acc[t] + buf[base + u, pl.ds(t * L, L)]
                            for t in range(HV))
            return acc
        acc0 = tuple(accv[pl.ds(t * L, L)] for t in range(HV))
        accf = lax.fori_loop(0, G // RU, _rows, acc0)
        for t in range(HV):
            accv[pl.ds(t * L, L)] = accf[t]

    def _accum_tail(k):
        buf = rings[k % 2]

        def _row(r, acc):
            return tuple(acc[t] + buf[r, pl.ds(t * L, L)]
                         for t in range(HV))
        acc0 = tuple(accv[pl.ds(t * L, L)] for t in range(HV))
        accf = lax.fori_loop(0, rem, _row, acc0)
        for t in range(HV):
            accv[pl.ds(t * L, L)] = accf[t]

    @pl.when(jnp.logical_and(0 < nrows, False))
    def _p0():
        _start(0)
    for k in range(0):
        if k + 1 < NBLK:
            @pl.when((k + 1) * G < nrows)
            def _st(k=k):
                _start(k + 1)

        @pl.when(k < nfull)
        def _af(k=k):
            _wait(k)
            _accum_full(k)

        @pl.when(jnp.logical_and(k == nfull, rem > 0))
        def _at(k=k):
            _wait(k)
            _accum_tail(k)

    ctot = jnp.zeros((L,), jnp.float32) + nrows.astype(jnp.float32)
    for t in range(HV):
        sl = pl.ds(t * L, L)
        accv[sl] = accv[sl] / ctot
    pltpu.sync_copy(accv, out_hbm.at[batch, pl.ds(h * HD, HD)])


_sc_kernel = functools.partial(
    pl.kernel,
    mesh=plsc.VectorSubcoreMesh(core_axis_name="c", subcore_axis_name="s"),
    out_type=jax.ShapeDtypeStruct((B, D), jnp.float32),
    compiler_params=pltpu.CompilerParams(needs_layout_passes=False),
    scratch_types=[
        pltpu.VMEM((N,), jnp.int32),
        pltpu.VMEM((IW,), jnp.int32),
        pltpu.VMEM((G, HD), jnp.float32),
        pltpu.VMEM((G, HD), jnp.float32),
        pltpu.VMEM((HD,), jnp.float32),
        pltpu.SemaphoreType.DMA,
        pltpu.SemaphoreType.DMA,
    ],
)(_sc_body)


def kernel(inputs, mask):
    x_half = inputs.reshape(B * N * 2, HD)
    m_i32 = mask.astype(jnp.int32).reshape(B * N)
    return _sc_kernel(x_half, m_i32)


# speedup vs baseline: 1.3234x; 1.3234x over previous
import functools
import jax
import jax.numpy as jnp
from jax import lax
from jax.experimental import pallas as pl
from jax.experimental.pallas import tpu as pltpu
from jax.experimental.pallas import tpu_sc as plsc

B, N, D = 16, 4096, 256
L = 16
NC, NS = 2, 16
BS = 8
BT = B - BS
WPB = (NC * NS) // BS
BSC = BS // NC
CPW = N // WPB
G = 128
NBLK = CPW // G
DV = D // L
CU = 8
RU = 8
TRASH = CPW + G
IW = TRASH + L


def _sc_body(x_hbm, mask_hbm, out_hbm, mvec, idxv, ring0, ring1, accv,
             tmpv, shsum, sem0, sem1):
    c = lax.axis_index("c")
    s = lax.axis_index("s")
    batch = c * BSC + s // WPB
    q = s % WPB
    row0 = batch * N

    pltpu.sync_copy(mask_hbm.at[pl.ds(row0, N)], mvec)

    def _count(i, tv):
        for j in range(CU):
            tv = tv + mvec[pl.ds((i * CU + j) * L, L)]
        return tv
    cnt_pl = lax.fori_loop(0, N // L // CU, _count, jnp.zeros((L,), jnp.int32))
    cnt_all = jnp.sum(cnt_pl)

    splat15 = jnp.full((L,), L - 1, jnp.int32)
    iota1 = lax.iota(jnp.int32, L)
    base = q * CPW

    def _compact(i, tv):
        for j in range(4):
            k = i * 4 + j
            mi = mvec[pl.ds(base + k * L, L)]
            cs = plsc.cumsum(mi)
            ids = iota1 + (row0 + base + k * L)
            pos = (tv + cs - 1) * mi + TRASH * (1 - mi)
            plsc.store_scatter(idxv, [pos], ids)
            tv = tv + cs[splat15]
        return tv
    tv = lax.fori_loop(0, CPW // L // 4, _compact, jnp.zeros((L,), jnp.int32))
    nrows = jnp.sum(tv) >> 4

    fillv = jnp.zeros((L,), jnp.int32) + row0
    for j in range(G // L):
        idxv[pl.ds(nrows + j * L, L)] = fillv

    for t in range(DV):
        accv[pl.ds(t * L, L)] = jnp.zeros((L,), jnp.float32)

    rings = (ring0, ring1)
    sems = (sem0, sem1)
    nfull = nrows // G
    rem = nrows - nfull * G

    def _start(k):
        pltpu.async_copy(x_hbm.at[idxv.at[pl.ds(k * G, G)]], rings[k % 2],
                         sems[k % 2])

    def _wait(k):
        pltpu.make_async_copy(x_hbm.at[idxv.at[pl.ds(k * G, G)]],
                              rings[k % 2], sems[k % 2]).wait()

    def _accum_full(k):
        buf = rings[k % 2]

        def _rows(r, acc):
            rb = r * RU
            for u in range(RU):
                acc = tuple(acc[t] + buf[rb + u, pl.ds(t * L, L)]
                            for t in range(DV))
            return acc
        acc0 = tuple(accv[pl.ds(t * L, L)] for t in range(DV))
        accf = lax.fori_loop(0, G // RU, _rows, acc0)
        for t in range(DV):
            accv[pl.ds(t * L, L)] = accf[t]

    def _accum_tail(k):
        buf = rings[k % 2]

        def _row(r, acc):
            return tuple(acc[t] + buf[r, pl.ds(t * L, L)]
                         for t in range(DV))
        acc0 = tuple(accv[pl.ds(t * L, L)] for t in range(DV))
        accf = lax.fori_loop(0, rem, _row, acc0)
        for t in range(DV):
            accv[pl.ds(t * L, L)] = accf[t]

    @pl.when(0 < nrows)
    def _p0():
        _start(0)
    for k in range(NBLK):
        if k + 1 < NBLK:
            @pl.when((k + 1) * G < nrows)
            def _st(k=k):
                _start(k + 1)

        @pl.when(k < nfull)
        def _af(k=k):
            _wait(k)
            _accum_full(k)

        @pl.when(jnp.logical_and(k == nfull, rem > 0))
        def _at(k=k):
            _wait(k)
            _accum_tail(k)

    pltpu.sync_copy(accv, shsum.at[s])
    plsc.subcore_barrier()

    @pl.when(q == 0)
    def _fin():
        for w in range(1, WPB):
            pltpu.sync_copy(shsum.at[s + w], tmpv)
            for t in range(DV):
                sl = pl.ds(t * L, L)
                accv[sl] = accv[sl] + tmpv[sl]
        ctot = jnp.zeros((L,), jnp.float32) + cnt_all.astype(jnp.float32)
        for t in range(DV):
            sl = pl.ds(t * L, L)
            accv[sl] = accv[sl] / ctot
        pltpu.sync_copy(accv, out_hbm.at[batch])


_sc_kernel = functools.partial(
    pl.kernel,
    mesh=plsc.VectorSubcoreMesh(core_axis_name="c", subcore_axis_name="s"),
    out_type=jax.ShapeDtypeStruct((BS, D), jnp.float32),
    compiler_params=pltpu.CompilerParams(needs_layout_passes=False),
    scratch_types=[
        pltpu.VMEM((N,), jnp.int32),
        pltpu.VMEM((IW,), jnp.int32),
        pltpu.VMEM((G, D), jnp.float32),
        pltpu.VMEM((G, D), jnp.float32),
        pltpu.VMEM((D,), jnp.float32),
        pltpu.VMEM((D,), jnp.float32),
        pltpu.VMEM_SHARED((NS, D), jnp.float32),
        pltpu.SemaphoreType.DMA,
        pltpu.SemaphoreType.DMA,
    ],
)(_sc_body)


TC_CH = 2048
TC_NCH = N // TC_CH


def _tc_body(x_ref, m_ref, o_ref, cnt_ref):
    j = pl.program_id(1)
    m = m_ref[0, 0, :]
    s = jnp.sum(x_ref[0] * m[:, None], axis=0)
    cntc = jnp.sum(m)

    @pl.when(j == 0)
    def _init():
        o_ref[0, 0, :] = s
        cnt_ref[0, 0] = cntc

    @pl.when(j > 0)
    def _acc():
        o_ref[0, 0, :] += s
        cnt_ref[0, 0] += cntc

    @pl.when(j == TC_NCH - 1)
    def _fin():
        o_ref[0, 0, :] = o_ref[0, 0, :] / cnt_ref[0, 0]


def kernel(inputs, mask):
    x_flat = inputs.reshape(B * N, D)
    m_i32 = mask.astype(jnp.int32).reshape(B * N)
    sc_out = _sc_kernel(x_flat, m_i32)

    m3 = mask.astype(jnp.float32).reshape(B, 1, N)
    tc_out = pl.pallas_call(
        _tc_body,
        grid=(BT, TC_NCH),
        in_specs=[
            pl.BlockSpec((1, TC_CH, D), lambda b, j: (b + BS, j, 0)),
            pl.BlockSpec((1, 1, TC_CH), lambda b, j: (b + BS, 0, j)),
        ],
        out_specs=pl.BlockSpec((1, 1, D), lambda b, j: (b, 0, 0)),
        out_shape=jax.ShapeDtypeStruct((BT, 1, D), jnp.float32),
        scratch_shapes=[pltpu.SMEM((1, 1), jnp.float32)],
    )(inputs, m3)
    return jnp.concatenate([sc_out, tc_out.reshape(BT, D)], axis=0)
